# baseline (speedup 1.0000x reference)
import functools

import jax
import jax.numpy as jnp
from jax.experimental import pallas as pl
from jax.experimental.pallas import tpu as pltpu

BATCH = 4096
IN_DIM = 128
INTER_DIM = 128
VOCAB = 1000
CH = 8
CPER = 16
CMAX = 1536
BP = BATCH + CH
RM = 512


def _chunk_matvec_body(widx_ref, rstart_ref, *refs):
    dw_refs = refs[:CPER]
    c_ref, out_ref = refs[CPER:]
    i = pl.program_id(0)
    for j in range(CPER):
        w = dw_refs[j][0].astype(jnp.bfloat16)
        k = rstart_ref[i * CPER + j]
        rows = c_ref[pl.ds(k, CH), :].astype(jnp.bfloat16)
        out_ref[pl.ds(k, CH), :] = jnp.tanh(
            jax.lax.dot(rows, w, preferred_element_type=jnp.float32))


def _logits_body(inter_ref, lw_ref, b_ref, out_ref):
    logits = jax.lax.dot_general(
        inter_ref[...].astype(jnp.bfloat16), lw_ref[...],
        (((1,), (1,)), ((), ())),
        preferred_element_type=jnp.float32)
    out_ref[...] = jax.nn.sigmoid(logits + b_ref[...])


@jax.jit
def kernel(vocab_ids, compressed, decoder_weights, linear_w, linear_b):
    counts = jnp.zeros((VOCAB,), jnp.int32).at[vocab_ids].add(1)
    csum = jnp.cumsum(counts, dtype=jnp.int32)
    starts = jnp.concatenate([jnp.zeros((1,), jnp.int32), csum])
    perm = jnp.argsort(vocab_ids)
    inv_perm = jnp.zeros((BATCH,), jnp.int32).at[perm].set(
        jnp.arange(BATCH, dtype=jnp.int32))
    c_sorted = jnp.zeros((BP, IN_DIM), jnp.float32).at[:BATCH].set(
        compressed[perm])

    nc = (counts + (CH - 1)) // CH
    ncum = jnp.cumsum(nc, dtype=jnp.int32)
    total = ncum[VOCAB - 1]
    c_idx = jnp.arange(CMAX, dtype=jnp.int32)
    g = jnp.searchsorted(ncum, c_idx, side='right').astype(jnp.int32)
    g = jnp.minimum(g, VOCAB - 1)
    m = c_idx - (ncum[g] - nc[g])
    real = c_idx < total
    row_start = jnp.where(real, starts[g] + CH * m, BATCH)
    last_g = g[jnp.maximum(total - 1, 0)]
    w_idx = jnp.where(real, g, last_g)

    def dw_index(i, widx, rst, j):
        return (widx[i * CPER + j], 0, 0)

    in_specs = [
        pl.BlockSpec((1, IN_DIM, INTER_DIM), functools.partial(dw_index, j=j))
        for j in range(CPER)
    ]
    in_specs.append(pl.BlockSpec((BP, IN_DIM), lambda i, widx, rst: (0, 0)))

    inter_sorted = pl.pallas_call(
        _chunk_matvec_body,
        grid_spec=pltpu.PrefetchScalarGridSpec(
            num_scalar_prefetch=2,
            grid=(CMAX // CPER,),
            in_specs=in_specs,
            out_specs=pl.BlockSpec((BP, INTER_DIM), lambda i, widx, rst: (0, 0)),
        ),
        out_shape=jax.ShapeDtypeStruct((BP, INTER_DIM), jnp.float32),
    )(w_idx, row_start, *([decoder_weights] * CPER), c_sorted)

    inter = inter_sorted[:BATCH][inv_perm]

    out = pl.pallas_call(
        _logits_body,
        grid=(BATCH // RM,),
        in_specs=[
            pl.BlockSpec((RM, INTER_DIM), lambda i: (i, 0)),
            pl.BlockSpec((VOCAB, INTER_DIM), lambda i: (0, 0)),
            pl.BlockSpec((1, VOCAB), lambda i: (0, 0)),
        ],
        out_specs=pl.BlockSpec((RM, VOCAB), lambda i: (i, 0)),
        out_shape=jax.ShapeDtypeStruct((BATCH, VOCAB), jnp.float32),
    )(inter, linear_w.astype(jnp.bfloat16), linear_b.reshape(1, VOCAB))
    return out


# speedup vs baseline: 1.5862x; 1.5862x over previous
import functools

import jax
import jax.numpy as jnp
from jax.experimental import pallas as pl
from jax.experimental.pallas import tpu as pltpu

BATCH = 4096
IN_DIM = 128
INTER_DIM = 128
VOCAB = 1000
CH = 8
CPER = 16
CMAX = 1536
BP = BATCH + CH
RM = 512


def _chunk_matvec_body(widx_ref, rstart_ref, *refs):
    dw_refs = refs[:CPER]
    c_ref, out_ref = refs[CPER:]
    i = pl.program_id(0)
    for j in range(CPER):
        w = dw_refs[j][0].astype(jnp.bfloat16)
        k = rstart_ref[i * CPER + j]
        rows = c_ref[pl.ds(k, CH), :].astype(jnp.bfloat16)
        out_ref[pl.ds(k, CH), :] = jnp.tanh(
            jax.lax.dot(rows, w, preferred_element_type=jnp.float32))


def _logits_body(inter_ref, lw_ref, b_ref, out_ref):
    logits = jax.lax.dot_general(
        inter_ref[...].astype(jnp.bfloat16), lw_ref[...],
        (((1,), (1,)), ((), ())),
        preferred_element_type=jnp.float32)
    out_ref[...] = jax.nn.sigmoid(logits + b_ref[...])


@jax.jit
def kernel(vocab_ids, compressed, decoder_weights, linear_w, linear_b):
    counts = jnp.zeros((VOCAB,), jnp.int32).at[vocab_ids].add(1)
    csum = jnp.cumsum(counts, dtype=jnp.int32)
    starts = jnp.concatenate([jnp.zeros((1,), jnp.int32), csum])
    perm = jnp.argsort(vocab_ids)
    inv_perm = jnp.zeros((BATCH,), jnp.int32).at[perm].set(
        jnp.arange(BATCH, dtype=jnp.int32))
    c_sorted = jnp.zeros((BP, IN_DIM), jnp.float32).at[:BATCH].set(
        compressed[perm])

    fixed_w = vocab_ids[perm[::CH]]
    fixed_k = jnp.arange(BATCH // CH, dtype=jnp.int32) * CH
    bound_w = jnp.arange(VOCAB, dtype=jnp.int32)
    bound_k = starts[:VOCAB]
    npad = CMAX - (BATCH // CH) - VOCAB
    w_idx = jnp.concatenate(
        [fixed_w, bound_w, jnp.full((npad,), VOCAB - 1, jnp.int32)])
    row_start = jnp.concatenate(
        [fixed_k, bound_k, jnp.full((npad,), BATCH, jnp.int32)])

    def dw_index(i, widx, rst, j):
        return (widx[i * CPER + j], 0, 0)

    in_specs = [
        pl.BlockSpec((1, IN_DIM, INTER_DIM), functools.partial(dw_index, j=j))
        for j in range(CPER)
    ]
    in_specs.append(pl.BlockSpec((BP, IN_DIM), lambda i, widx, rst: (0, 0)))

    inter_sorted = pl.pallas_call(
        _chunk_matvec_body,
        grid_spec=pltpu.PrefetchScalarGridSpec(
            num_scalar_prefetch=2,
            grid=(CMAX // CPER,),
            in_specs=in_specs,
            out_specs=pl.BlockSpec((BP, INTER_DIM), lambda i, widx, rst: (0, 0)),
        ),
        out_shape=jax.ShapeDtypeStruct((BP, INTER_DIM), jnp.float32),
    )(w_idx, row_start, *([decoder_weights] * CPER), c_sorted)

    inter = inter_sorted[:BATCH][inv_perm]

    out = pl.pallas_call(
        _logits_body,
        grid=(BATCH // RM,),
        in_specs=[
            pl.BlockSpec((RM, INTER_DIM), lambda i: (i, 0)),
            pl.BlockSpec((VOCAB, INTER_DIM), lambda i: (0, 0)),
            pl.BlockSpec((1, VOCAB), lambda i: (0, 0)),
        ],
        out_specs=pl.BlockSpec((RM, VOCAB), lambda i: (i, 0)),
        out_shape=jax.ShapeDtypeStruct((BATCH, VOCAB), jnp.float32),
    )(inter, linear_w.astype(jnp.bfloat16), linear_b.reshape(1, VOCAB))
    return out


# speedup vs baseline: 1.7378x; 1.0956x over previous
import functools

import jax
import jax.numpy as jnp
from jax import lax
from jax.experimental import pallas as pl
from jax.experimental.pallas import tpu as pltpu
from jax.experimental.pallas import tpu_sc as plsc

BATCH = 4096
IN_DIM = 128
INTER_DIM = 128
VOCAB = 1000
CH = 8
CPER = 16
CMAX = 1536
BP = BATCH + CH
RM = 512


def _chunk_matvec_body(widx_ref, rstart_ref, *refs):
    dw_refs = refs[:CPER]
    c_ref, out_ref = refs[CPER:]
    i = pl.program_id(0)
    for j in range(CPER):
        w = dw_refs[j][0].astype(jnp.bfloat16)
        k = rstart_ref[i * CPER + j]
        rows = c_ref[pl.ds(k, CH), :].astype(jnp.bfloat16)
        out_ref[pl.ds(k, CH), :] = jnp.tanh(
            jax.lax.dot(rows, w, preferred_element_type=jnp.float32))


L = 16
VP = 1024


def _sc_sort_body(ids_hbm, perm_hbm, inv_hbm, starts_hbm, whead_hbm,
                  ids_v, hist_v, perm_v, inv_v, starts_v, whead_v):
    wid = lax.axis_index("s") * 2 + lax.axis_index("c")

    @pl.when(wid == 0)
    def _():
        pltpu.sync_copy(ids_hbm, ids_v)
        iota = lax.iota(jnp.int32, L)
        zeros = jnp.zeros((L,), jnp.int32)
        ones = jnp.ones((L,), jnp.int32)

        def zero_body(k, c):
            hist_v[pl.ds(k * L, L)] = zeros
            return c

        lax.fori_loop(0, (L * VP) // L, zero_body, 0)

        def hist_body(c, carry):
            vec = ids_v[pl.ds(c * L, L)]
            plsc.addupdate_scatter(hist_v, [vec + iota * VP], ones)
            return carry

        lax.fori_loop(0, BATCH // L, hist_body, 0)

        def merge_body(k, carry):
            acc = hist_v[pl.ds(k * L, L)]
            for l in range(1, L):
                acc = acc + hist_v[pl.ds(l * VP + k * L, L)]
            incl = plsc.cumsum(acc)
            svec = incl - acc + carry
            starts_v[pl.ds(k * L, L)] = svec
            base = svec
            for l in range(L):
                tmp = hist_v[pl.ds(l * VP + k * L, L)]
                hist_v[pl.ds(l * VP + k * L, L)] = base
                base = base + tmp
            return carry + jnp.sum(acc)

        lax.fori_loop(0, VP // L, merge_body, 0)

        def place_body(c, carry):
            vec = ids_v[pl.ds(c * L, L)]
            idx = vec + iota * VP
            cur = plsc.load_gather(hist_v, [idx])
            rows = c * L + iota
            plsc.store_scatter(perm_v, [cur], rows)
            inv_v[pl.ds(c * L, L)] = cur
            plsc.store_scatter(hist_v, [idx], cur + 1)
            head = (cur & 7) == 0
            plsc.store_scatter(whead_v, [lax.shift_right_logical(cur, 3)],
                               vec, mask=head)
            return carry

        lax.fori_loop(0, BATCH // L, place_body, 0)

        pltpu.sync_copy(perm_v, perm_hbm)
        pltpu.sync_copy(inv_v, inv_hbm)
        pltpu.sync_copy(starts_v, starts_hbm)
        pltpu.sync_copy(whead_v, whead_hbm)


@functools.partial(
    pl.kernel,
    out_type=[
        jax.ShapeDtypeStruct((BATCH,), jnp.int32),
        jax.ShapeDtypeStruct((BATCH,), jnp.int32),
        jax.ShapeDtypeStruct((VP,), jnp.int32),
        jax.ShapeDtypeStruct((BATCH // CH,), jnp.int32),
    ],
    mesh=plsc.VectorSubcoreMesh(core_axis_name="c", subcore_axis_name="s"),
    compiler_params=pltpu.CompilerParams(needs_layout_passes=False),
    scratch_types=[
        pltpu.VMEM((BATCH,), jnp.int32),
        pltpu.VMEM((L * VP,), jnp.int32),
        pltpu.VMEM((BATCH,), jnp.int32),
        pltpu.VMEM((BATCH,), jnp.int32),
        pltpu.VMEM((VP,), jnp.int32),
        pltpu.VMEM((BATCH // CH,), jnp.int32),
    ],
)
def _sc_sort(ids_hbm, perm_hbm, inv_hbm, starts_hbm, whead_hbm,
             ids_v, hist_v, perm_v, inv_v, starts_v, whead_v):
    _sc_sort_body(ids_hbm, perm_hbm, inv_hbm, starts_hbm, whead_hbm,
                  ids_v, hist_v, perm_v, inv_v, starts_v, whead_v)


def _logits_body(inter_ref, lw_ref, b_ref, out_ref):
    logits = jax.lax.dot_general(
        inter_ref[...].astype(jnp.bfloat16), lw_ref[...],
        (((1,), (1,)), ((), ())),
        preferred_element_type=jnp.float32)
    out_ref[...] = jax.nn.sigmoid(logits + b_ref[...])


@jax.jit
def kernel(vocab_ids, compressed, decoder_weights, linear_w, linear_b):
    perm, inv_perm, starts, fixed_w = _sc_sort(vocab_ids)
    c_sorted = jnp.zeros((BP, IN_DIM), jnp.float32).at[:BATCH].set(
        compressed[perm])

    fixed_k = jnp.arange(BATCH // CH, dtype=jnp.int32) * CH
    bound_w = jnp.arange(VOCAB, dtype=jnp.int32)
    bound_k = starts[:VOCAB]
    npad = CMAX - (BATCH // CH) - VOCAB
    w_idx = jnp.concatenate(
        [fixed_w, bound_w, jnp.full((npad,), VOCAB - 1, jnp.int32)])
    row_start = jnp.concatenate(
        [fixed_k, bound_k, jnp.full((npad,), BATCH, jnp.int32)])

    def dw_index(i, widx, rst, j):
        return (widx[i * CPER + j], 0, 0)

    in_specs = [
        pl.BlockSpec((1, IN_DIM, INTER_DIM), functools.partial(dw_index, j=j))
        for j in range(CPER)
    ]
    in_specs.append(pl.BlockSpec((BP, IN_DIM), lambda i, widx, rst: (0, 0)))

    inter_sorted = pl.pallas_call(
        _chunk_matvec_body,
        grid_spec=pltpu.PrefetchScalarGridSpec(
            num_scalar_prefetch=2,
            grid=(CMAX // CPER,),
            in_specs=in_specs,
            out_specs=pl.BlockSpec((BP, INTER_DIM), lambda i, widx, rst: (0, 0)),
        ),
        out_shape=jax.ShapeDtypeStruct((BP, INTER_DIM), jnp.float32),
    )(w_idx, row_start, *([decoder_weights] * CPER), c_sorted)

    inter = inter_sorted[:BATCH][inv_perm]

    out = pl.pallas_call(
        _logits_body,
        grid=(BATCH // RM,),
        in_specs=[
            pl.BlockSpec((RM, INTER_DIM), lambda i: (i, 0)),
            pl.BlockSpec((VOCAB, INTER_DIM), lambda i: (0, 0)),
            pl.BlockSpec((1, VOCAB), lambda i: (0, 0)),
        ],
        out_specs=pl.BlockSpec((RM, VOCAB), lambda i: (i, 0)),
        out_shape=jax.ShapeDtypeStruct((BATCH, VOCAB), jnp.float32),
    )(inter, linear_w.astype(jnp.bfloat16), linear_b.reshape(1, VOCAB))
    return out
